# baseline (speedup 1.0000x reference)
import functools

import jax
import jax.numpy as jnp
from jax.experimental import pallas as pl

N1 = 32768
N2 = 8192
B = 8
CIN = 512
COUT = 256
K = 3

QB = 2048
N1B = N1 // B
N2B = N2 // B
QBLKS = N1B // QB

ABLK = 8192
BBLK = 8192


def _linear_stats_kernel(x_ref, w_ref, b_ref, y_ref, s_ref, ss_ref):
    i = pl.program_id(0)
    y = jnp.dot(x_ref[...], w_ref[...], preferred_element_type=jnp.float32)
    y = y + b_ref[...]
    y_ref[...] = y
    cs = jnp.sum(y, axis=0, keepdims=True)
    css = jnp.sum(y * y, axis=0, keepdims=True)

    @pl.when(i == 0)
    def _():
        s_ref[...] = cs
        ss_ref[...] = css

    @pl.when(i != 0)
    def _():
        s_ref[...] += cs
        ss_ref[...] += css


def _linear_stats(x, w, b, blk):
    n, cin = x.shape
    cout = w.shape[1]
    grid = (n // blk,)
    return pl.pallas_call(
        _linear_stats_kernel,
        grid=grid,
        in_specs=[
            pl.BlockSpec((blk, cin), lambda i: (i, 0)),
            pl.BlockSpec((cin, cout), lambda i: (0, 0)),
            pl.BlockSpec((1, cout), lambda i: (0, 0)),
        ],
        out_specs=[
            pl.BlockSpec((blk, cout), lambda i: (i, 0)),
            pl.BlockSpec((1, cout), lambda i: (0, 0)),
            pl.BlockSpec((1, cout), lambda i: (0, 0)),
        ],
        out_shape=[
            jax.ShapeDtypeStruct((n, cout), jnp.float32),
            jax.ShapeDtypeStruct((1, cout), jnp.float32),
            jax.ShapeDtypeStruct((1, cout), jnp.float32),
        ],
    )(x, w, b)


def _triple_insert(a1, a2, a3, x):
    m = jnp.minimum(a1, x)
    M = jnp.maximum(a1, x)
    m2 = jnp.minimum(a2, M)
    M2 = jnp.maximum(a2, M)
    return m, m2, jnp.minimum(a3, M2)


def _triple_merge(a1, a2, a3, b1, b2, b3):
    M1 = jnp.maximum(a1, b1)
    c1 = jnp.minimum(a1, b1)
    m2 = jnp.minimum(a2, b2)
    c2 = jnp.minimum(M1, m2)
    c3 = jnp.minimum(jnp.minimum(jnp.maximum(m2, M1), jnp.maximum(a2, b2)),
                     jnp.minimum(a3, b3))
    return c1, c2, c3


def _fused_kernel(y1_ref, q_ref, p_ref, q2_ref, y2_ref,
                  sc1_ref, sh1_ref, sc2_ref, sh2_ref,
                  out_ref):
    f1 = jnp.maximum(y1_ref[...] * sc1_ref[...] + sh1_ref[...], 0.0)
    f2 = jnp.maximum(y2_ref[...] * sc2_ref[...] + sh2_ref[...], 0.0)

    qa = q_ref[...]
    pa = p_ref[...]
    q2 = q2_ref[0]
    p2 = pa[:, 3:4]
    crossT = jax.lax.dot_general(
        pa, qa, (((1,), (1,)), ((), ())),
        preferred_element_type=jnp.float32)
    dT = jnp.maximum((q2 + p2) + crossT, 0.0)

    a1 = dT[0:128]
    a2 = jnp.full_like(a1, jnp.inf)
    a3 = a2
    for i in range(1, 8):
        a1, a2, a3 = _triple_insert(a1, a2, a3, dT[128 * i:128 * (i + 1)])
    h = 64
    while h >= 1:
        a1, a2, a3 = _triple_merge(a1[0:h], a2[0:h], a3[0:h],
                                   a1[h:2 * h], a2[h:2 * h], a3[h:2 * h])
        h //= 2
    inv_norm = 1.0 / (1.0 / (a1 + 1e-8) + 1.0 / (a2 + 1e-8) + 1.0 / (a3 + 1e-8))

    r = 1.0 / (dT + 1e-8)
    w = jnp.where(dT <= a3, r, 0.0) * inv_norm

    interp = jax.lax.dot_general(
        w, f2, (((0,), (0,)), ((), ())),
        preferred_element_type=jnp.float32)
    out_ref[...] = f1 + interp


def _bn_affine(s, ss, n, g, be):
    mean = s[0] / n
    var = ss[0] / n - mean * mean
    scale = g * jax.lax.rsqrt(var + 1e-5)
    shift = be - mean * scale
    return scale.reshape(1, -1), shift.reshape(1, -1)


@jax.jit
def _run(point_1, feat_1, point_2, feat_2, W1, b1, g1, be1, W2, b2, g2, be2):
    q2 = jnp.sum(point_1 * point_1, axis=1)
    p2 = jnp.sum(point_2 * point_2, axis=1, keepdims=True)
    q2t = q2.reshape(N1 // QB, 1, QB)
    q_pad = jnp.pad(-2.0 * point_1, ((0, 0), (0, 5)))
    p_pad = jnp.concatenate(
        [point_2, p2, jnp.zeros((N2, 4), jnp.float32)], axis=1)

    y1, s1, ss1 = _linear_stats(feat_1, W1, b1.reshape(1, -1), ABLK)
    y2, s2, ss2 = _linear_stats(feat_2, W2, b2.reshape(1, -1), BBLK)

    sc1, sh1 = _bn_affine(s1, ss1, N1, g1, be1)
    sc2, sh2 = _bn_affine(s2, ss2, N2, g2, be2)

    vec = pl.BlockSpec((1, COUT), lambda b, j: (0, 0))
    out = pl.pallas_call(
        _fused_kernel,
        grid=(B, QBLKS),
        in_specs=[
            pl.BlockSpec((QB, COUT), lambda b, j: (b * QBLKS + j, 0)),
            pl.BlockSpec((QB, 8), lambda b, j: (b * QBLKS + j, 0)),
            pl.BlockSpec((N2B, 8), lambda b, j: (b, 0)),
            pl.BlockSpec((1, 1, QB), lambda b, j: (b * QBLKS + j, 0, 0)),
            pl.BlockSpec((N2B, COUT), lambda b, j: (b, 0)),
            vec, vec, vec, vec,
        ],
        out_specs=pl.BlockSpec((QB, COUT), lambda b, j: (b * QBLKS + j, 0)),
        out_shape=jax.ShapeDtypeStruct((N1, COUT), jnp.float32),
    )(y1, q_pad, p_pad, q2t, y2, sc1, sh1, sc2, sh2)
    return out


def kernel(point_1, feat_1, point_2, feat_2, W1, b1, g1, be1, W2, b2, g2, be2,
           row_splits_1, row_splits_2):
    return _run(point_1, feat_1, point_2, feat_2,
                W1, b1, g1, be1, W2, b2, g2, be2)


# speedup vs baseline: 1.0427x; 1.0427x over previous
import functools

import jax
import jax.numpy as jnp
from jax.experimental import pallas as pl

N1 = 32768
N2 = 8192
B = 8
CIN = 512
COUT = 256
K = 3

QB = 2048
N1B = N1 // B
N2B = N2 // B
QBLKS = N1B // QB

ABLK = 4096
BBLK = 4096


def _linear_stats_kernel(x_ref, w_ref, b_ref, y_ref, s_ref, ss_ref):
    i = pl.program_id(0)
    y = jnp.dot(x_ref[...], w_ref[...], preferred_element_type=jnp.float32)
    y = y + b_ref[...]
    y_ref[...] = y
    cs = jnp.sum(y, axis=0, keepdims=True)
    css = jnp.sum(y * y, axis=0, keepdims=True)

    @pl.when(i == 0)
    def _():
        s_ref[...] = cs
        ss_ref[...] = css

    @pl.when(i != 0)
    def _():
        s_ref[...] += cs
        ss_ref[...] += css


def _linear_stats(x, w, b, blk):
    n, cin = x.shape
    cout = w.shape[1]
    grid = (n // blk,)
    return pl.pallas_call(
        _linear_stats_kernel,
        grid=grid,
        in_specs=[
            pl.BlockSpec((blk, cin), lambda i: (i, 0)),
            pl.BlockSpec((cin, cout), lambda i: (0, 0)),
            pl.BlockSpec((1, cout), lambda i: (0, 0)),
        ],
        out_specs=[
            pl.BlockSpec((blk, cout), lambda i: (i, 0)),
            pl.BlockSpec((1, cout), lambda i: (0, 0)),
            pl.BlockSpec((1, cout), lambda i: (0, 0)),
        ],
        out_shape=[
            jax.ShapeDtypeStruct((n, cout), jnp.float32),
            jax.ShapeDtypeStruct((1, cout), jnp.float32),
            jax.ShapeDtypeStruct((1, cout), jnp.float32),
        ],
    )(x, w, b)


def _triple_insert(a1, a2, a3, x):
    m = jnp.minimum(a1, x)
    M = jnp.maximum(a1, x)
    m2 = jnp.minimum(a2, M)
    M2 = jnp.maximum(a2, M)
    return m, m2, jnp.minimum(a3, M2)


def _triple_merge(a1, a2, a3, b1, b2, b3):
    M1 = jnp.maximum(a1, b1)
    c1 = jnp.minimum(a1, b1)
    m2 = jnp.minimum(a2, b2)
    c2 = jnp.minimum(M1, m2)
    c3 = jnp.minimum(jnp.minimum(jnp.maximum(m2, M1), jnp.maximum(a2, b2)),
                     jnp.minimum(a3, b3))
    return c1, c2, c3


def _fused_kernel(y1_ref, q_ref, p_ref, q2_ref, y2_ref,
                  sc1_ref, sh1_ref, sc2_ref, sh2_ref,
                  out_ref):
    f1 = jnp.maximum(y1_ref[...] * sc1_ref[...] + sh1_ref[...], 0.0)
    f2 = jnp.maximum(y2_ref[...] * sc2_ref[...] + sh2_ref[...], 0.0)

    qa = q_ref[...]
    pa = p_ref[...]
    q2 = q2_ref[0]
    p2 = pa[:, 3:4]
    crossT = jax.lax.dot_general(
        pa, qa, (((1,), (1,)), ((), ())),
        preferred_element_type=jnp.float32)
    dT = jnp.maximum((q2 + p2) + crossT, 0.0)

    a1 = dT[0:128]
    a2 = jnp.full_like(a1, jnp.inf)
    a3 = a2
    for i in range(1, 8):
        a1, a2, a3 = _triple_insert(a1, a2, a3, dT[128 * i:128 * (i + 1)])
    h = 64
    while h >= 1:
        a1, a2, a3 = _triple_merge(a1[0:h], a2[0:h], a3[0:h],
                                   a1[h:2 * h], a2[h:2 * h], a3[h:2 * h])
        h //= 2
    inv_norm = 1.0 / (1.0 / (a1 + 1e-8) + 1.0 / (a2 + 1e-8) + 1.0 / (a3 + 1e-8))

    r = 1.0 / (dT + 1e-8)
    w = jnp.where(dT <= a3, r, 0.0) * inv_norm

    interp = jax.lax.dot_general(
        w.astype(jnp.bfloat16), f2.astype(jnp.bfloat16),
        (((0,), (0,)), ((), ())),
        preferred_element_type=jnp.float32)
    out_ref[...] = f1 + interp


def _bn_affine(s, ss, n, g, be):
    mean = s[0] / n
    var = ss[0] / n - mean * mean
    scale = g * jax.lax.rsqrt(var + 1e-5)
    shift = be - mean * scale
    return scale.reshape(1, -1), shift.reshape(1, -1)


@jax.jit
def _run(point_1, feat_1, point_2, feat_2, W1, b1, g1, be1, W2, b2, g2, be2):
    q2 = jnp.sum(point_1 * point_1, axis=1)
    p2 = jnp.sum(point_2 * point_2, axis=1, keepdims=True)
    q2t = q2.reshape(N1 // QB, 1, QB)
    q_pad = jnp.pad(-2.0 * point_1, ((0, 0), (0, 5)))
    p_pad = jnp.concatenate(
        [point_2, p2, jnp.zeros((N2, 4), jnp.float32)], axis=1)

    y1, s1, ss1 = _linear_stats(feat_1, W1, b1.reshape(1, -1), ABLK)
    y2, s2, ss2 = _linear_stats(feat_2, W2, b2.reshape(1, -1), BBLK)

    sc1, sh1 = _bn_affine(s1, ss1, N1, g1, be1)
    sc2, sh2 = _bn_affine(s2, ss2, N2, g2, be2)

    vec = pl.BlockSpec((1, COUT), lambda b, j: (0, 0))
    out = pl.pallas_call(
        _fused_kernel,
        grid=(B, QBLKS),
        in_specs=[
            pl.BlockSpec((QB, COUT), lambda b, j: (b * QBLKS + j, 0)),
            pl.BlockSpec((QB, 8), lambda b, j: (b * QBLKS + j, 0)),
            pl.BlockSpec((N2B, 8), lambda b, j: (b, 0)),
            pl.BlockSpec((1, 1, QB), lambda b, j: (b * QBLKS + j, 0, 0)),
            pl.BlockSpec((N2B, COUT), lambda b, j: (b, 0)),
            vec, vec, vec, vec,
        ],
        out_specs=pl.BlockSpec((QB, COUT), lambda b, j: (b * QBLKS + j, 0)),
        out_shape=jax.ShapeDtypeStruct((N1, COUT), jnp.float32),
    )(y1, q_pad, p_pad, q2t, y2, sc1, sh1, sc2, sh2)
    return out


def kernel(point_1, feat_1, point_2, feat_2, W1, b1, g1, be1, W2, b2, g2, be2,
           row_splits_1, row_splits_2):
    return _run(point_1, feat_1, point_2, feat_2,
                W1, b1, g1, be1, W2, b2, g2, be2)


# speedup vs baseline: 1.0448x; 1.0020x over previous
import functools

import jax
import jax.numpy as jnp
from jax.experimental import pallas as pl

N1 = 32768
N2 = 8192
B = 8
CIN = 512
COUT = 256
K = 3

QB = 2048
N1B = N1 // B
N2B = N2 // B
QBLKS = N1B // QB

ABLK = 4096
BBLK = 4096


def _linear_stats_kernel(x_ref, w_ref, b_ref, y_ref, s_ref, ss_ref):
    i = pl.program_id(0)
    y = jnp.dot(x_ref[...], w_ref[...], preferred_element_type=jnp.float32)
    y = y + b_ref[...]
    y_ref[...] = y
    cs = jnp.sum(y, axis=0, keepdims=True)
    css = jnp.sum(y * y, axis=0, keepdims=True)

    @pl.when(i == 0)
    def _():
        s_ref[...] = cs
        ss_ref[...] = css

    @pl.when(i != 0)
    def _():
        s_ref[...] += cs
        ss_ref[...] += css


def _linear_stats(x, w, b, blk):
    n, cin = x.shape
    cout = w.shape[1]
    grid = (n // blk,)
    return pl.pallas_call(
        _linear_stats_kernel,
        grid=grid,
        in_specs=[
            pl.BlockSpec((blk, cin), lambda i: (i, 0)),
            pl.BlockSpec((cin, cout), lambda i: (0, 0)),
            pl.BlockSpec((1, cout), lambda i: (0, 0)),
        ],
        out_specs=[
            pl.BlockSpec((blk, cout), lambda i: (i, 0)),
            pl.BlockSpec((1, cout), lambda i: (0, 0)),
            pl.BlockSpec((1, cout), lambda i: (0, 0)),
        ],
        out_shape=[
            jax.ShapeDtypeStruct((n, cout), jnp.float32),
            jax.ShapeDtypeStruct((1, cout), jnp.float32),
            jax.ShapeDtypeStruct((1, cout), jnp.float32),
        ],
    )(x, w, b)


def _triple_insert(a1, a2, a3, x):
    m = jnp.minimum(a1, x)
    M = jnp.maximum(a1, x)
    m2 = jnp.minimum(a2, M)
    M2 = jnp.maximum(a2, M)
    return m, m2, jnp.minimum(a3, M2)


def _triple_merge(a1, a2, a3, b1, b2, b3):
    M1 = jnp.maximum(a1, b1)
    c1 = jnp.minimum(a1, b1)
    m2 = jnp.minimum(a2, b2)
    c2 = jnp.minimum(M1, m2)
    c3 = jnp.minimum(jnp.minimum(jnp.maximum(m2, M1), jnp.maximum(a2, b2)),
                     jnp.minimum(a3, b3))
    return c1, c2, c3


def _fused_kernel(y1_ref, q_ref, p_ref, q2_ref, y2_ref,
                  sc1_ref, sh1_ref, sc2_ref, sh2_ref,
                  out_ref):
    f1 = jnp.maximum(y1_ref[...] * sc1_ref[...] + sh1_ref[...], 0.0)
    f2 = jnp.maximum(y2_ref[...] * sc2_ref[...] + sh2_ref[...], 0.0)

    qa = q_ref[...]
    pa = p_ref[...]
    q2 = q2_ref[0]
    p2 = pa[:, 3:4]
    crossT = jax.lax.dot_general(
        pa, qa, (((1,), (1,)), ((), ())),
        preferred_element_type=jnp.float32)
    dT = jnp.maximum((q2 + p2) + crossT, 0.0)

    c = [dT[128 * i:128 * (i + 1)] for i in range(8)]
    lo = [jnp.minimum(c[2 * i], c[2 * i + 1]) for i in range(4)]
    hi = [jnp.maximum(c[2 * i], c[2 * i + 1]) for i in range(4)]
    tri = []
    for i in range(2):
        a1, a2 = lo[2 * i], hi[2 * i]
        b1, b2 = lo[2 * i + 1], hi[2 * i + 1]
        M1 = jnp.maximum(a1, b1)
        t1 = jnp.minimum(a1, b1)
        m2 = jnp.minimum(a2, b2)
        t2 = jnp.minimum(M1, m2)
        t3 = jnp.minimum(jnp.maximum(m2, M1), jnp.maximum(a2, b2))
        tri.append((t1, t2, t3))
    a1, a2, a3 = _triple_merge(*tri[0], *tri[1])
    h = 64
    while h >= 1:
        a1, a2, a3 = _triple_merge(a1[0:h], a2[0:h], a3[0:h],
                                   a1[h:2 * h], a2[h:2 * h], a3[h:2 * h])
        h //= 2
    inv_norm = 1.0 / (1.0 / (a1 + 1e-8) + 1.0 / (a2 + 1e-8) + 1.0 / (a3 + 1e-8))

    r = 1.0 / (dT + 1e-8)
    w = jnp.where(dT <= a3, r, 0.0) * inv_norm

    interp = jax.lax.dot_general(
        w.astype(jnp.bfloat16), f2.astype(jnp.bfloat16),
        (((0,), (0,)), ((), ())),
        preferred_element_type=jnp.float32)
    out_ref[...] = f1 + interp


def _bn_affine(s, ss, n, g, be):
    mean = s[0] / n
    var = ss[0] / n - mean * mean
    scale = g * jax.lax.rsqrt(var + 1e-5)
    shift = be - mean * scale
    return scale.reshape(1, -1), shift.reshape(1, -1)


@jax.jit
def _run(point_1, feat_1, point_2, feat_2, W1, b1, g1, be1, W2, b2, g2, be2):
    q2 = jnp.sum(point_1 * point_1, axis=1)
    p2 = jnp.sum(point_2 * point_2, axis=1, keepdims=True)
    q2t = q2.reshape(N1 // QB, 1, QB)
    q_pad = jnp.pad(-2.0 * point_1, ((0, 0), (0, 5)))
    p_pad = jnp.concatenate(
        [point_2, p2, jnp.zeros((N2, 4), jnp.float32)], axis=1)

    y1, s1, ss1 = _linear_stats(feat_1, W1, b1.reshape(1, -1), ABLK)
    y2, s2, ss2 = _linear_stats(feat_2, W2, b2.reshape(1, -1), BBLK)

    sc1, sh1 = _bn_affine(s1, ss1, N1, g1, be1)
    sc2, sh2 = _bn_affine(s2, ss2, N2, g2, be2)

    vec = pl.BlockSpec((1, COUT), lambda b, j: (0, 0))
    out = pl.pallas_call(
        _fused_kernel,
        grid=(B, QBLKS),
        in_specs=[
            pl.BlockSpec((QB, COUT), lambda b, j: (b * QBLKS + j, 0)),
            pl.BlockSpec((QB, 8), lambda b, j: (b * QBLKS + j, 0)),
            pl.BlockSpec((N2B, 8), lambda b, j: (b, 0)),
            pl.BlockSpec((1, 1, QB), lambda b, j: (b * QBLKS + j, 0, 0)),
            pl.BlockSpec((N2B, COUT), lambda b, j: (b, 0)),
            vec, vec, vec, vec,
        ],
        out_specs=pl.BlockSpec((QB, COUT), lambda b, j: (b * QBLKS + j, 0)),
        out_shape=jax.ShapeDtypeStruct((N1, COUT), jnp.float32),
    )(y1, q_pad, p_pad, q2t, y2, sc1, sh1, sc2, sh2)
    return out


def kernel(point_1, feat_1, point_2, feat_2, W1, b1, g1, be1, W2, b2, g2, be2,
           row_splits_1, row_splits_2):
    return _run(point_1, feat_1, point_2, feat_2,
                W1, b1, g1, be1, W2, b2, g2, be2)


# speedup vs baseline: 1.0771x; 1.0309x over previous
import functools

import jax
import jax.numpy as jnp
from jax.experimental import pallas as pl

N1 = 32768
N2 = 8192
B = 8
CIN = 512
COUT = 256
K = 3

QB = 2048
N1B = N1 // B
N2B = N2 // B
QBLKS = N1B // QB

ABLK = 4096
BBLK = 4096


def _linear_stats_kernel(store_y, x_ref, w_ref, b_ref, *out_refs):
    i = pl.program_id(0)
    y = jnp.dot(x_ref[...], w_ref[...], preferred_element_type=jnp.float32)
    y = y + b_ref[...]
    if store_y:
        y_ref, s_ref, ss_ref = out_refs
        y_ref[...] = y
    else:
        s_ref, ss_ref = out_refs
    cs = jnp.sum(y, axis=0, keepdims=True)
    css = jnp.sum(y * y, axis=0, keepdims=True)

    @pl.when(i == 0)
    def _():
        s_ref[...] = cs
        ss_ref[...] = css

    @pl.when(i != 0)
    def _():
        s_ref[...] += cs
        ss_ref[...] += css


def _linear_stats(x, w, b, blk, store_y=True):
    n, cin = x.shape
    cout = w.shape[1]
    grid = (n // blk,)
    stat_spec = pl.BlockSpec((1, cout), lambda i: (0, 0))
    stat_shape = jax.ShapeDtypeStruct((1, cout), jnp.float32)
    out_specs = [stat_spec, stat_spec]
    out_shape = [stat_shape, stat_shape]
    if store_y:
        out_specs.insert(0, pl.BlockSpec((blk, cout), lambda i: (i, 0)))
        out_shape.insert(0, jax.ShapeDtypeStruct((n, cout), jnp.float32))
    return pl.pallas_call(
        functools.partial(_linear_stats_kernel, store_y),
        grid=grid,
        in_specs=[
            pl.BlockSpec((blk, cin), lambda i: (i, 0)),
            pl.BlockSpec((cin, cout), lambda i: (0, 0)),
            pl.BlockSpec((1, cout), lambda i: (0, 0)),
        ],
        out_specs=out_specs,
        out_shape=out_shape,
    )(x, w, b)


def _triple_insert(a1, a2, a3, x):
    m = jnp.minimum(a1, x)
    M = jnp.maximum(a1, x)
    m2 = jnp.minimum(a2, M)
    M2 = jnp.maximum(a2, M)
    return m, m2, jnp.minimum(a3, M2)


def _triple_merge(a1, a2, a3, b1, b2, b3):
    M1 = jnp.maximum(a1, b1)
    c1 = jnp.minimum(a1, b1)
    m2 = jnp.minimum(a2, b2)
    c2 = jnp.minimum(M1, m2)
    c3 = jnp.minimum(jnp.minimum(jnp.maximum(m2, M1), jnp.maximum(a2, b2)),
                     jnp.minimum(a3, b3))
    return c1, c2, c3


def _fused_kernel(x1_ref, w1_ref, b1_ref, q_ref, p_ref, q2_ref, y2_ref,
                  sc1_ref, sh1_ref, sc2_ref, sh2_ref,
                  out_ref):
    y1 = jnp.dot(x1_ref[...], w1_ref[...],
                 preferred_element_type=jnp.float32) + b1_ref[...]
    f1 = jnp.maximum(y1 * sc1_ref[...] + sh1_ref[...], 0.0)
    f2 = jnp.maximum(y2_ref[...] * sc2_ref[...] + sh2_ref[...], 0.0)

    qa = q_ref[...]
    pa = p_ref[...]
    q2 = q2_ref[0]
    p2 = pa[:, 3:4]
    crossT = jax.lax.dot_general(
        pa, qa, (((1,), (1,)), ((), ())),
        preferred_element_type=jnp.float32)
    dT = jnp.maximum((q2 + p2) + crossT, 0.0)

    c = [dT[128 * i:128 * (i + 1)] for i in range(8)]
    lo = [jnp.minimum(c[2 * i], c[2 * i + 1]) for i in range(4)]
    hi = [jnp.maximum(c[2 * i], c[2 * i + 1]) for i in range(4)]
    tri = []
    for i in range(2):
        a1, a2 = lo[2 * i], hi[2 * i]
        b1, b2 = lo[2 * i + 1], hi[2 * i + 1]
        M1 = jnp.maximum(a1, b1)
        t1 = jnp.minimum(a1, b1)
        m2 = jnp.minimum(a2, b2)
        t2 = jnp.minimum(M1, m2)
        t3 = jnp.minimum(jnp.maximum(m2, M1), jnp.maximum(a2, b2))
        tri.append((t1, t2, t3))
    a1, a2, a3 = _triple_merge(*tri[0], *tri[1])
    h = 64
    while h >= 1:
        a1, a2, a3 = _triple_merge(a1[0:h], a2[0:h], a3[0:h],
                                   a1[h:2 * h], a2[h:2 * h], a3[h:2 * h])
        h //= 2
    inv_norm = 1.0 / (1.0 / (a1 + 1e-8) + 1.0 / (a2 + 1e-8) + 1.0 / (a3 + 1e-8))

    r = 1.0 / (dT + 1e-8)
    w = jnp.where(dT <= a3, r, 0.0) * inv_norm

    interp = jax.lax.dot_general(
        w.astype(jnp.bfloat16), f2.astype(jnp.bfloat16),
        (((0,), (0,)), ((), ())),
        preferred_element_type=jnp.float32)
    out_ref[...] = f1 + interp


def _bn_affine(s, ss, n, g, be):
    mean = s[0] / n
    var = ss[0] / n - mean * mean
    scale = g * jax.lax.rsqrt(var + 1e-5)
    shift = be - mean * scale
    return scale.reshape(1, -1), shift.reshape(1, -1)


@jax.jit
def _run(point_1, feat_1, point_2, feat_2, W1, b1, g1, be1, W2, b2, g2, be2):
    q2 = jnp.sum(point_1 * point_1, axis=1)
    p2 = jnp.sum(point_2 * point_2, axis=1, keepdims=True)
    q2t = q2.reshape(N1 // QB, 1, QB)
    q_pad = jnp.pad(-2.0 * point_1, ((0, 0), (0, 5)))
    p_pad = jnp.concatenate(
        [point_2, p2, jnp.zeros((N2, 4), jnp.float32)], axis=1)

    s1, ss1 = _linear_stats(feat_1, W1, b1.reshape(1, -1), ABLK,
                            store_y=False)
    y2, s2, ss2 = _linear_stats(feat_2, W2, b2.reshape(1, -1), BBLK)

    sc1, sh1 = _bn_affine(s1, ss1, N1, g1, be1)
    sc2, sh2 = _bn_affine(s2, ss2, N2, g2, be2)

    vec = pl.BlockSpec((1, COUT), lambda b, j: (0, 0))
    out = pl.pallas_call(
        _fused_kernel,
        grid=(B, QBLKS),
        in_specs=[
            pl.BlockSpec((QB, COUT), lambda b, j: (b * QBLKS + j, 0)),
            pl.BlockSpec((COUT, COUT), lambda b, j: (0, 0)),
            vec,
            pl.BlockSpec((QB, 8), lambda b, j: (b * QBLKS + j, 0)),
            pl.BlockSpec((N2B, 8), lambda b, j: (b, 0)),
            pl.BlockSpec((1, 1, QB), lambda b, j: (b * QBLKS + j, 0, 0)),
            pl.BlockSpec((N2B, COUT), lambda b, j: (b, 0)),
            vec, vec, vec, vec,
        ],
        out_specs=pl.BlockSpec((QB, COUT), lambda b, j: (b * QBLKS + j, 0)),
        out_shape=jax.ShapeDtypeStruct((N1, COUT), jnp.float32),
    )(feat_1, W1, b1.reshape(1, -1), q_pad, p_pad, q2t, y2, sc1, sh1, sc2, sh2)
    return out


def kernel(point_1, feat_1, point_2, feat_2, W1, b1, g1, be1, W2, b2, g2, be2,
           row_splits_1, row_splits_2):
    return _run(point_1, feat_1, point_2, feat_2,
                W1, b1, g1, be1, W2, b2, g2, be2)
